# baseline (speedup 1.0000x reference)
import jax
import jax.numpy as jnp
from jax import lax
from jax.experimental import pallas as pl
from jax.experimental.pallas import tpu as pltpu
from jax.experimental.pallas import tpu_sc as plsc

_INFO = plsc.get_sparse_core_info()
_NC, _NS, _L = _INFO.num_cores, _INFO.num_subcores, _INFO.num_lanes
_NW = _NC * _NS

_B = 16384
_D = 64
_V = 1000000
_CW = 512
_NCH = 122
_VMAIN = 16 * _NCH * _CW
_TAILW = 640
_PITCH = 645
_LCAP = 1664
_CCAP = 64
_DUMP = _B
_BPW = _B // _NW

_MESH = plsc.VectorSubcoreMesh(core_axis_name="c", subcore_axis_name="s")


def _prefilter(idx_hbm, idx_vm, lidx, lpos, lo, hi, iota16):
    def init_body(v, c):
        lidx[pl.ds(v * _L, _L)] = jnp.full((_L,), -1, jnp.int32)
        lpos[pl.ds(v * _L, _L)] = _DUMP + iota16
        return c

    lax.fori_loop(0, _LCAP // _L, init_body, 0, unroll=8)

    off = 0
    for p in range(4):
        pltpu.sync_copy(idx_hbm.at[pl.ds(p * (_B // 4), _B // 4)], idx_vm)

        def scan_body(v, off, _p=p):
            iv = idx_vm[pl.ds(v * _L, _L)]
            pv = iota16 + (_p * (_B // 4) + v * _L)
            m = (iv >= lo) & (iv < hi)
            plsc.store_compressed(lidx.at[pl.ds(off, _L)], iv, mask=m)
            plsc.store_compressed(lpos.at[pl.ds(off, _L)], pv, mask=m)
            return off + plsc.all_reduce_population_count(m)[0]

        off = lax.fori_loop(0, _B // 4 // _L, scan_body, off, unroll=8)
    return off


def _rescan(lidx, lpos, c0, c1, crm, cpos, iota16):
    def init_body(v, c):
        crm[pl.ds(v * _L, _L)] = jnp.zeros((_L,), jnp.int32)
        cpos[pl.ds(v * _L, _L)] = _DUMP + iota16
        return c

    lax.fori_loop(0, _CCAP // _L, init_body, 0, unroll=4)

    def scan_body(v, off):
        iv = lidx[pl.ds(v * _L, _L)]
        pv = lpos[pl.ds(v * _L, _L)]
        m = (iv >= c0) & (iv < c1)
        plsc.store_compressed(crm.at[pl.ds(off, _L)], iv - c0, mask=m)
        plsc.store_compressed(cpos.at[pl.ds(off, _L)], pv, mask=m)
        return off + plsc.all_reduce_population_count(m)[0]

    return lax.fori_loop(0, _LCAP // _L, scan_body, 0, unroll=8)


def _scan_table(idx_hbm, tabT_hbm, tail_hbm, stage_hbm, s, scale, wch,
                idx_vm, stage, lidx, lpos, crm, cpos, obuf, opos, sem, osem,
                iota16):
    lo = s * (_NCH * _CW)
    hi = jnp.where(s == _NS - 1, _V, lo + _NCH * _CW)

    _prefilter(idx_hbm, idx_vm, lidx, lpos, lo, hi, iota16)

    def fire(c, slot):
        is_tail = (s == _NS - 1) & (c == _NCH)
        col0 = lo + c * _CW

        @pl.when(c < _NCH)
        def _():
            for g in range(8):
                pltpu.async_copy(
                    tabT_hbm.at[pl.ds(g * 8, 8), pl.ds(col0, _CW)],
                    stage.at[slot, pl.ds(g * 8, 8), pl.ds(0, _CW)], sem)

        @pl.when(is_tail)
        def _():
            pltpu.async_copy(tail_hbm, stage.at[slot, :, pl.ds(0, _TAILW)],
                             sem)

    def drain(c, slot):
        is_tail = (s == _NS - 1) & (c == _NCH)

        @pl.when(c < _NCH)
        def _():
            for g in range(8):
                pltpu.make_async_copy(
                    tabT_hbm.at[pl.ds(g * 8, 8), pl.ds(0, _CW)],
                    stage.at[slot, pl.ds(g * 8, 8), pl.ds(0, _CW)],
                    sem).wait()

        @pl.when(is_tail)
        def _():
            pltpu.make_async_copy(
                tail_hbm, stage.at[slot, :, pl.ds(0, _TAILW)], sem).wait()

    fire(0, 0)

    def chunk_body(c, carry):
        is_tail = (s == _NS - 1) & (c == _NCH)
        active = (c < _NCH) | is_tail
        slot = lax.rem(c, 2)

        fire(c + 1, 1 - slot)
        drain(c, slot)

        @pl.when(active)
        def _():
            col0 = lo + c * _CW
            c0 = jnp.where(is_tail, _VMAIN, col0)
            c1 = jnp.where(is_tail, _V, col0 + _CW)

            nc = _rescan(lidx, lpos, c0, c1, crm, cpos, iota16)

            @pl.when(c >= 2)
            def _():
                pltpu.make_async_copy(
                    obuf.at[slot], stage_hbm.at[opos.at[slot]], osem).wait()

            def elem_body(e, c2):
                rm = crm[pl.ds(e, _L)][0]
                for ci in range(4):
                    rows = iota16 + ci * _L
                    col = jnp.full((_L,), 0, jnp.int32) + rm
                    vals = plsc.load_gather(stage.at[slot], [rows, col])
                    if scale:
                        vals = vals * wch[ci]
                    obuf[slot, e, pl.ds(ci * _L, _L)] = vals
                return c2

            lax.fori_loop(0, nc, elem_body, 0)
            for v in range(_CCAP // _L):
                opos[slot, pl.ds(v * _L, _L)] = cpos[pl.ds(v * _L, _L)]
            pltpu.async_copy(obuf.at[slot], stage_hbm.at[opos.at[slot]], osem)

        return carry

    lax.fori_loop(0, _NCH + 1, chunk_body, 0)
    for slot in range(2):
        pltpu.make_async_copy(obuf.at[slot], stage_hbm.at[opos.at[slot]],
                              osem).wait()


def _scan_body(uidx_hbm, iidx_hbm, utabT_hbm, itabT_hbm, tailu_hbm,
               taili_hbm, w_hbm,
               ustage_hbm, istage_hbm,
               idx_vm, w_v, stage, lidx, lpos, crm, cpos, obuf, opos,
               sem, osem):
    sc = lax.axis_index("c")
    s = lax.axis_index("s")
    iota16 = lax.iota(jnp.int32, _L)
    pltpu.sync_copy(w_hbm, w_v)
    wch = [w_v[pl.ds(ci * _L, _L)] for ci in range(4)]

    @pl.when(sc == 0)
    def _():
        _scan_table(uidx_hbm, utabT_hbm, tailu_hbm, ustage_hbm, s, True, wch,
                    idx_vm, stage, lidx, lpos, crm, cpos, obuf, opos,
                    sem, osem, iota16)

    @pl.when(sc == 1)
    def _():
        _scan_table(iidx_hbm, itabT_hbm, taili_hbm, istage_hbm, s, False, wch,
                    idx_vm, stage, lidx, lpos, crm, cpos, obuf, opos,
                    sem, osem, iota16)


def _combine_body(ustage_hbm, istage_hbm, b_hbm, out_hbm,
                  ubuf, ibuf, b_v, accbuf, outbuf, sem):
    wid = lax.axis_index("s") * _NC + lax.axis_index("c")
    base = wid * _BPW
    iota16 = lax.iota(jnp.int32, _L)
    pltpu.sync_copy(b_hbm, b_v)
    bias = b_v[...]

    def quarter_body(q, carry):
        rb = base + q * 128
        pltpu.sync_copy(ustage_hbm.at[pl.ds(rb, 128)], ubuf)
        pltpu.sync_copy(istage_hbm.at[pl.ds(rb, 128)], ibuf)

        def row_body(k, c2):
            acc = (ubuf[k, pl.ds(0, _L)] * ibuf[k, pl.ds(0, _L)]
                   + ubuf[k, pl.ds(_L, _L)] * ibuf[k, pl.ds(_L, _L)]
                   + ubuf[k, pl.ds(2 * _L, _L)] * ibuf[k, pl.ds(2 * _L, _L)]
                   + ubuf[k, pl.ds(3 * _L, _L)] * ibuf[k, pl.ds(3 * _L, _L)])
            accbuf[pl.ds((q * 128 + k) * 17, _L)] = acc
            return c2

        lax.fori_loop(0, 128, row_body, 0, unroll=4)
        return carry

    lax.fori_loop(0, _BPW // 128, quarter_body, 0)

    def grp_body(g, carry):
        flat = iota16 * 17 + g * (_L * 17)
        acc = bias
        for l in range(_L):
            acc = acc + plsc.load_gather(accbuf, [flat + l])
        outbuf[pl.ds(g * _L, _L)] = 1.0 / (1.0 + jnp.exp(-acc))
        return carry

    lax.fori_loop(0, _BPW // _L, grp_body, 0)
    pltpu.sync_copy(outbuf, out_hbm.at[pl.ds(base, _BPW)])


@jax.jit
def _gmf_call(uidx, iidx, utabT, itabT, tailu, taili, w_flat, b_vec):
    ustage, istage = pl.kernel(
        _scan_body,
        mesh=_MESH,
        out_type=(jax.ShapeDtypeStruct((_B + _L, 128), jnp.float32),
                  jax.ShapeDtypeStruct((_B + _L, 128), jnp.float32)),
        scratch_types=[
            pltpu.VMEM((_B // 4,), jnp.int32),
            pltpu.VMEM((_D,), jnp.float32),
            pltpu.VMEM((2, _D, _PITCH), jnp.float32),
            pltpu.VMEM((_LCAP,), jnp.int32),
            pltpu.VMEM((_LCAP,), jnp.int32),
            pltpu.VMEM((_CCAP + _L,), jnp.int32),
            pltpu.VMEM((_CCAP + _L,), jnp.int32),
            pltpu.VMEM((2, _CCAP, 128), jnp.float32),
            pltpu.VMEM((2, _CCAP), jnp.int32),
            pltpu.SemaphoreType.DMA,
            pltpu.SemaphoreType.DMA,
        ],
        compiler_params=pltpu.CompilerParams(needs_layout_passes=False),
    )(uidx, iidx, utabT, itabT, tailu, taili, w_flat)

    out = pl.kernel(
        _combine_body,
        mesh=_MESH,
        out_type=jax.ShapeDtypeStruct((_B,), jnp.float32),
        scratch_types=[
            pltpu.VMEM((128, 128), jnp.float32),
            pltpu.VMEM((128, 128), jnp.float32),
            pltpu.VMEM((_L,), jnp.float32),
            pltpu.VMEM((_BPW * 17,), jnp.float32),
            pltpu.VMEM((_BPW,), jnp.float32),
            pltpu.SemaphoreType.DMA,
        ],
        compiler_params=pltpu.CompilerParams(needs_layout_passes=False),
    )(ustage, istage, b_vec)
    return out


def kernel(user_indices, item_indices, user_table, item_table, affine_w,
           affine_b):
    uidx = user_indices.astype(jnp.int32)
    iidx = item_indices.astype(jnp.int32)
    utabT = user_table.T
    itabT = item_table.T
    tailu = jnp.pad(utabT[:, _VMAIN:], ((0, 0), (0, _TAILW - (_V - _VMAIN))))
    taili = jnp.pad(itabT[:, _VMAIN:], ((0, 0), (0, _TAILW - (_V - _VMAIN))))
    w_flat = affine_w.reshape(_D)
    b_vec = jnp.broadcast_to(affine_b.reshape(()), (_L,))
    out = _gmf_call(uidx, iidx, utabT, itabT, tailu, taili, w_flat, b_vec)
    return out.reshape(_B, 1)


# speedup vs baseline: 1.5964x; 1.5964x over previous
import jax
import jax.numpy as jnp
from jax import lax
from jax.experimental import pallas as pl
from jax.experimental.pallas import tpu as pltpu
from jax.experimental.pallas import tpu_sc as plsc

_INFO = plsc.get_sparse_core_info()
_NC, _NS, _L = _INFO.num_cores, _INFO.num_subcores, _INFO.num_lanes
_NW = _NC * _NS

_B = 16384
_D = 64
_V = 1000000
_CW = 1024
_NCH = 61
_VMAIN = 16 * _NCH * _CW
_TAILW = 640
_PITCH = 1029
_LCAP = 1664
_CCAP = 64
_DUMP = _B
_BPW = _B // _NW

_MESH = plsc.VectorSubcoreMesh(core_axis_name="c", subcore_axis_name="s")


def _prefilter(idx_hbm, idx_vm, lidx, lpos, lo, hi, iota16):
    def init_body(v, c):
        lidx[pl.ds(v * _L, _L)] = jnp.full((_L,), -1, jnp.int32)
        lpos[pl.ds(v * _L, _L)] = _DUMP + iota16
        return c

    lax.fori_loop(0, _LCAP // _L, init_body, 0, unroll=8)

    off = 0
    for p in range(4):
        pltpu.sync_copy(idx_hbm.at[pl.ds(p * (_B // 4), _B // 4)], idx_vm)

        def scan_body(v, off, _p=p):
            iv = idx_vm[pl.ds(v * _L, _L)]
            pv = iota16 + (_p * (_B // 4) + v * _L)
            m = (iv >= lo) & (iv < hi)
            plsc.store_compressed(lidx.at[pl.ds(off, _L)], iv, mask=m)
            plsc.store_compressed(lpos.at[pl.ds(off, _L)], pv, mask=m)
            return off + plsc.all_reduce_population_count(m)[0]

        off = lax.fori_loop(0, _B // 4 // _L, scan_body, off, unroll=8)
    return off


def _rescan(lidx, lpos, c0, c1, crm, cpos, iota16, nlv):
    def init_body(v, c):
        crm[pl.ds(v * _L, _L)] = jnp.zeros((_L,), jnp.int32)
        cpos[pl.ds(v * _L, _L)] = _DUMP + iota16
        return c

    lax.fori_loop(0, _CCAP // _L, init_body, 0, unroll=4)

    def scan_body(v, off):
        iv = lidx[pl.ds(v * _L, _L)]
        pv = lpos[pl.ds(v * _L, _L)]
        m = (iv >= c0) & (iv < c1)
        plsc.store_compressed(crm.at[pl.ds(off, _L)], iv - c0, mask=m)
        plsc.store_compressed(cpos.at[pl.ds(off, _L)], pv, mask=m)
        return off + plsc.all_reduce_population_count(m)[0]

    return lax.fori_loop(0, nlv, scan_body, 0)


def _scan_table(idx_hbm, tabT_hbm, tail_hbm, stage_hbm, s, scale,
                idx_vm, w_v, stage, lidx, lpos, crm, cpos, obuf, opos,
                sem, osem, iota16):
    lo = s * (_NCH * _CW)
    hi = jnp.where(s == _NS - 1, _V, lo + _NCH * _CW)

    nl = _prefilter(idx_hbm, idx_vm, lidx, lpos, lo, hi, iota16)
    nlv = (nl + _L - 1) // _L

    def fire(f):
        c = f // 2
        h = f % 2
        is_tail = (s == _NS - 1) & (c == _NCH)
        col0 = lo + c * _CW

        @pl.when(c < _NCH)
        def _():
            for g in range(4):
                pltpu.async_copy(
                    tabT_hbm.at[pl.ds((4 * h + g) * 8, 8), pl.ds(col0, _CW)],
                    stage.at[h, pl.ds(g * 8, 8), pl.ds(0, _CW)], sem)

        @pl.when(is_tail)
        def _():
            pltpu.async_copy(tail_hbm.at[pl.ds(32 * h, 32), :],
                             stage.at[h, :, pl.ds(0, _TAILW)], sem)

    def drain(f):
        c = f // 2
        h = f % 2
        is_tail = (s == _NS - 1) & (c == _NCH)

        @pl.when(c < _NCH)
        def _():
            for g in range(4):
                pltpu.make_async_copy(
                    tabT_hbm.at[pl.ds(g * 8, 8), pl.ds(0, _CW)],
                    stage.at[h, pl.ds(g * 8, 8), pl.ds(0, _CW)], sem).wait()

        @pl.when(is_tail)
        def _():
            pltpu.make_async_copy(
                tail_hbm.at[pl.ds(0, 32), :],
                stage.at[h, :, pl.ds(0, _TAILW)], sem).wait()

    fire(0)

    def half_body(f, nc):
        c = f // 2
        h = f % 2
        sslot = lax.rem(c, 2)
        is_tail = (s == _NS - 1) & (c == _NCH)
        active = (c < _NCH) | is_tail

        fire(f + 1)
        drain(f)

        c0 = jnp.where(is_tail, _VMAIN, lo + c * _CW)
        c1 = jnp.where(is_tail, _V, c0 + _CW)

        def on_h0(nc_):
            @pl.when(c >= 2)
            def _():
                pltpu.make_async_copy(
                    obuf.at[sslot], stage_hbm.at[opos.at[sslot]], osem).wait()

            return _rescan(lidx, lpos, c0, c1, crm, cpos, iota16, nlv)

        nc = lax.cond(active & (h == 0), on_h0, lambda x: x, nc)

        @pl.when(active)
        def _():
            h32 = h * 32
            wc = [w_v[pl.ds(h32 + cil * _L, _L)] for cil in range(2)]

            def elem_body(e, c2):
                rm = crm[pl.ds(e, _L)][0]
                for cil in range(2):
                    rows = iota16 + cil * _L
                    col = jnp.full((_L,), 0, jnp.int32) + rm
                    vals = plsc.load_gather(stage.at[h], [rows, col])
                    if scale:
                        vals = vals * wc[cil]
                    obuf[sslot, e, pl.ds(h32 + cil * _L, _L)] = vals
                return c2

            lax.fori_loop(0, nc, elem_body, 0)

        @pl.when(active & (h == 1))
        def _():
            for v in range(_CCAP // _L):
                opos[sslot, pl.ds(v * _L, _L)] = cpos[pl.ds(v * _L, _L)]
            pltpu.async_copy(obuf.at[sslot], stage_hbm.at[opos.at[sslot]],
                             osem)

        return nc

    lax.fori_loop(0, 2 * (_NCH + 1), half_body, 0)
    for slot in range(2):
        pltpu.make_async_copy(obuf.at[slot], stage_hbm.at[opos.at[slot]],
                              osem).wait()


def _scan_body(uidx_hbm, iidx_hbm, utabT_hbm, itabT_hbm, tailu_hbm,
               taili_hbm, w_hbm,
               ustage_hbm, istage_hbm,
               idx_vm, w_v, stage, lidx, lpos, crm, cpos, obuf, opos,
               sem, osem):
    sc = lax.axis_index("c")
    s = lax.axis_index("s")
    iota16 = lax.iota(jnp.int32, _L)
    pltpu.sync_copy(w_hbm, w_v)

    @pl.when(sc == 0)
    def _():
        _scan_table(uidx_hbm, utabT_hbm, tailu_hbm, ustage_hbm, s, True,
                    idx_vm, w_v, stage, lidx, lpos, crm, cpos, obuf, opos,
                    sem, osem, iota16)

    @pl.when(sc == 1)
    def _():
        _scan_table(iidx_hbm, itabT_hbm, taili_hbm, istage_hbm, s, False,
                    idx_vm, w_v, stage, lidx, lpos, crm, cpos, obuf, opos,
                    sem, osem, iota16)


def _combine_body(ustage_hbm, istage_hbm, b_hbm, out_hbm,
                  ubuf, ibuf, b_v, accbuf, outbuf, sem):
    wid = lax.axis_index("s") * _NC + lax.axis_index("c")
    base = wid * _BPW
    iota16 = lax.iota(jnp.int32, _L)
    pltpu.sync_copy(b_hbm, b_v)
    bias = b_v[...]

    def quarter_body(q, carry):
        rb = base + q * 128
        pltpu.sync_copy(ustage_hbm.at[pl.ds(rb, 128)], ubuf)
        pltpu.sync_copy(istage_hbm.at[pl.ds(rb, 128)], ibuf)

        def row_body(k, c2):
            acc = (ubuf[k, pl.ds(0, _L)] * ibuf[k, pl.ds(0, _L)]
                   + ubuf[k, pl.ds(_L, _L)] * ibuf[k, pl.ds(_L, _L)]
                   + ubuf[k, pl.ds(2 * _L, _L)] * ibuf[k, pl.ds(2 * _L, _L)]
                   + ubuf[k, pl.ds(3 * _L, _L)] * ibuf[k, pl.ds(3 * _L, _L)])
            accbuf[pl.ds((q * 128 + k) * 17, _L)] = acc
            return c2

        lax.fori_loop(0, 128, row_body, 0, unroll=4)
        return carry

    lax.fori_loop(0, _BPW // 128, quarter_body, 0)

    def grp_body(g, carry):
        flat = iota16 * 17 + g * (_L * 17)
        acc = bias
        for l in range(_L):
            acc = acc + plsc.load_gather(accbuf, [flat + l])
        outbuf[pl.ds(g * _L, _L)] = 1.0 / (1.0 + jnp.exp(-acc))
        return carry

    lax.fori_loop(0, _BPW // _L, grp_body, 0)
    pltpu.sync_copy(outbuf, out_hbm.at[pl.ds(base, _BPW)])


@jax.jit
def _gmf_call(uidx, iidx, utabT, itabT, tailu, taili, w_flat, b_vec):
    ustage, istage = pl.kernel(
        _scan_body,
        mesh=_MESH,
        out_type=(jax.ShapeDtypeStruct((_B + _L, 128), jnp.float32),
                  jax.ShapeDtypeStruct((_B + _L, 128), jnp.float32)),
        scratch_types=[
            pltpu.VMEM((_B // 4,), jnp.int32),
            pltpu.VMEM((_D,), jnp.float32),
            pltpu.VMEM((2, 32, _PITCH), jnp.float32),
            pltpu.VMEM((_LCAP,), jnp.int32),
            pltpu.VMEM((_LCAP,), jnp.int32),
            pltpu.VMEM((_CCAP + _L,), jnp.int32),
            pltpu.VMEM((_CCAP + _L,), jnp.int32),
            pltpu.VMEM((2, _CCAP, 128), jnp.float32),
            pltpu.VMEM((2, _CCAP), jnp.int32),
            pltpu.SemaphoreType.DMA,
            pltpu.SemaphoreType.DMA,
        ],
        compiler_params=pltpu.CompilerParams(needs_layout_passes=False),
    )(uidx, iidx, utabT, itabT, tailu, taili, w_flat)

    out = pl.kernel(
        _combine_body,
        mesh=_MESH,
        out_type=jax.ShapeDtypeStruct((_B,), jnp.float32),
        scratch_types=[
            pltpu.VMEM((128, 128), jnp.float32),
            pltpu.VMEM((128, 128), jnp.float32),
            pltpu.VMEM((_L,), jnp.float32),
            pltpu.VMEM((_BPW * 17,), jnp.float32),
            pltpu.VMEM((_BPW,), jnp.float32),
            pltpu.SemaphoreType.DMA,
        ],
        compiler_params=pltpu.CompilerParams(needs_layout_passes=False),
    )(ustage, istage, b_vec)
    return out


def kernel(user_indices, item_indices, user_table, item_table, affine_w,
           affine_b):
    uidx = user_indices.astype(jnp.int32)
    iidx = item_indices.astype(jnp.int32)
    utabT = user_table.T
    itabT = item_table.T
    tailu = jnp.pad(utabT[:, _VMAIN:], ((0, 0), (0, _TAILW - (_V - _VMAIN))))
    taili = jnp.pad(itabT[:, _VMAIN:], ((0, 0), (0, _TAILW - (_V - _VMAIN))))
    w_flat = affine_w.reshape(_D)
    b_vec = jnp.broadcast_to(affine_b.reshape(()), (_L,))
    out = _gmf_call(uidx, iidx, utabT, itabT, tailu, taili, w_flat, b_vec)
    return out.reshape(_B, 1)
